# Initial kernel scaffold; baseline (speedup 1.0000x reference)
import functools

import jax
import jax.numpy as jnp
from jax import lax
from jax.experimental import pallas as pl
from jax.experimental.pallas import tpu as pltpu
from jax.experimental.pallas import tpu_sc as plsc

N = 10000
E = 320000
D = 128

NC = 2
NS = 16
NW = NC * NS
EPT = E // NW
CH = 80
NCHUNK = EPT // CH
RPT = N // NS

_mesh = plsc.VectorSubcoreMesh(core_axis_name="c", subcore_axis_name="s")


def _sc_deg_body(dst_hbm, w_hbm, out_hbm, dst_v, w_v, deg_v):
    c = lax.axis_index("c")
    s = lax.axis_index("s")
    wid = s * NC + c
    pltpu.sync_copy(dst_hbm.at[wid], dst_v)
    pltpu.sync_copy(w_hbm.at[wid], w_v)

    zeros = jnp.zeros((16,), jnp.float32)

    def zero_body(i, carry):
        deg_v[pl.ds(i * 16, 16)] = zeros
        return carry

    lax.fori_loop(0, N // 16, zero_body, 0)

    def body(i, carry):
        d = dst_v[pl.ds(i * 16, 16)]
        ww = w_v[pl.ds(i * 16, 16)]
        plsc.addupdate_scatter(deg_v, [d], ww)
        return carry

    lax.fori_loop(0, EPT // 16, body, 0)
    pltpu.sync_copy(deg_v, out_hbm.at[wid])


def _sc_deg(dst2, w2):
    k = functools.partial(
        pl.kernel,
        mesh=_mesh,
        out_type=jax.ShapeDtypeStruct((NW, N), jnp.float32),
        scratch_types=[
            pltpu.VMEM((EPT,), jnp.int32),
            pltpu.VMEM((EPT,), jnp.float32),
            pltpu.VMEM((N,), jnp.float32),
        ],
    )(_sc_deg_body)
    return k(dst2, w2)


def _sc_edge_body(src_hbm, dst_hbm, w_hbm, hs_hbm, out_hbm,
                  dst_v, w_v, src_v, rows_v, acc_sh, sem):
    c = lax.axis_index("c")
    s = lax.axis_index("s")
    wid = s * NC + c

    pltpu.sync_copy(dst_hbm.at[wid], dst_v)
    pltpu.sync_copy(w_hbm.at[wid], w_v)

    zeros = jnp.zeros((16,), jnp.float32)
    for i in range(CH):
        for j in range(D // 16):
            rows_v[i, pl.ds(j * 16, 16)] = zeros
    rbase = s * RPT
    for k in range(7):
        pltpu.sync_copy(rows_v, acc_sh.at[pl.ds(rbase + k * CH, CH)])
    pltpu.sync_copy(rows_v.at[pl.ds(0, RPT - 7 * CH)],
                    acc_sh.at[pl.ds(rbase + 7 * CH, RPT - 7 * CH)])
    plsc.subcore_barrier()

    def chunk_body(ci, carry):
        pltpu.sync_copy(src_hbm.at[wid, ci], src_v)
        pltpu.async_copy(hs_hbm.at[src_v], rows_v, sem).wait()
        for g in range(CH // 16):
            for r in range(16):
                e = g * 16 + r
                lane = jnp.full((16,), ci * CH + e, jnp.int32)
                ws = plsc.load_gather(w_v, [lane])
                for j in range(D // 16):
                    rows_v[e, pl.ds(j * 16, 16)] = (
                        rows_v[e, pl.ds(j * 16, 16)] * ws)
        pltpu.sync_copy(rows_v, acc_sh.at[dst_v.at[ci]], add=True)
        return carry

    lax.fori_loop(0, NCHUNK, chunk_body, 0)
    plsc.subcore_barrier()
    rbase = s * RPT
    pltpu.sync_copy(acc_sh.at[pl.ds(rbase, RPT)],
                    out_hbm.at[c, pl.ds(rbase, RPT)])


def _sc_edges(src3, dst3, w2, hs):
    k = functools.partial(
        pl.kernel,
        mesh=_mesh,
        out_type=jax.ShapeDtypeStruct((NC, N, D), jnp.float32),
        scratch_types=[
            pltpu.VMEM((NCHUNK, CH), jnp.int32),
            pltpu.VMEM((EPT,), jnp.float32),
            pltpu.VMEM((CH,), jnp.int32),
            pltpu.VMEM((CH, D), jnp.float32),
            pltpu.VMEM_SHARED((N, D), jnp.float32),
            pltpu.SemaphoreType.DMA,
        ],
    )(_sc_edge_body)
    return k(src3, dst3, w2, hs)


_BLK = 1000


def _tc_mm_body(x_ref, w_ref, degp_ref, h_ref, hs_ref):
    xb = x_ref[...]
    h = jnp.dot(xb, w_ref[...], preferred_element_type=jnp.float32)
    deg = 1.0 + jnp.sum(degp_ref[...], axis=1, keepdims=True)
    dinv = lax.rsqrt(deg)
    h_ref[...] = h
    hs_ref[...] = h * dinv


def _tc_mm(x, W, degp_t):
    grid = (N // _BLK,)
    return pl.pallas_call(
        _tc_mm_body,
        grid=grid,
        in_specs=[
            pl.BlockSpec((_BLK, D), lambda i: (i, 0)),
            pl.BlockSpec((D, D), lambda i: (0, 0)),
            pl.BlockSpec((_BLK, NW), lambda i: (i, 0)),
        ],
        out_specs=[
            pl.BlockSpec((_BLK, D), lambda i: (i, 0)),
            pl.BlockSpec((_BLK, D), lambda i: (i, 0)),
        ],
        out_shape=[
            jax.ShapeDtypeStruct((N, D), jnp.float32),
            jax.ShapeDtypeStruct((N, D), jnp.float32),
        ],
    )(x, W, degp_t)


def _tc_final_body(acc_ref, h_ref, degp_ref, b_ref, g_ref, be_ref, o_ref):
    acc = acc_ref[0] + acc_ref[1]
    deg = 1.0 + jnp.sum(degp_ref[...], axis=1, keepdims=True)
    dinv = lax.rsqrt(deg)
    pre = acc * dinv + h_ref[...] * (dinv * dinv) + b_ref[...]
    mean = jnp.mean(pre, axis=0, keepdims=True)
    var = jnp.mean((pre - mean) * (pre - mean), axis=0, keepdims=True)
    o = (pre - mean) * lax.rsqrt(var + 1e-5) * g_ref[...] + be_ref[...]
    o_ref[...] = jnp.maximum(o, 0.0)


def _tc_final(acc, h, degp_t, b, gamma, beta):
    return pl.pallas_call(
        _tc_final_body,
        out_shape=jax.ShapeDtypeStruct((N, D), jnp.float32),
    )(acc, h, degp_t, b, gamma, beta)


def kernel(x, edge_index, edge_weight, W, b, gamma, beta):
    src = edge_index[0]
    dst = edge_index[1]
    src3 = src.reshape(NW, NCHUNK, CH)
    dst3 = dst.reshape(NW, NCHUNK, CH)
    dst2 = dst.reshape(NW, EPT)
    w2 = edge_weight.reshape(NW, EPT)

    degp = _sc_deg(dst2, w2)
    degp_t = degp.T
    h, hs = _tc_mm(x, W, degp_t)
    acc = _sc_edges(src3, dst3, w2, hs)
    out = _tc_final(acc, h, degp_t,
                    b.reshape(1, D), gamma.reshape(1, D), beta.reshape(1, D))
    return out


# speedup vs baseline: 15.4071x; 15.4071x over previous
import functools

import jax
import jax.numpy as jnp
from jax import lax
from jax.experimental import pallas as pl
from jax.experimental.pallas import tpu as pltpu
from jax.experimental.pallas import tpu_sc as plsc

N = 10000
E = 320000
D = 128

NC = 2
NS = 16
NW = NC * NS
EPT = E // NW
CH = 80
NCHUNK = EPT // CH
RPT = N // NS

_mesh = plsc.VectorSubcoreMesh(core_axis_name="c", subcore_axis_name="s")

_GD = lax.GatherDimensionNumbers(
    offset_dims=(), collapsed_slice_dims=(0,), start_index_map=(0,))


def _splat(v, r):
    idx = jnp.full((16, 1), r, jnp.int32)
    return lax.gather(v, idx, _GD, (1,),
                      mode=lax.GatherScatterMode.PROMISE_IN_BOUNDS)


def _sc_deg_body(dst_hbm, w_hbm, out_hbm, dst_v, wchunk_v, zbuf, deg_sh):
    c = lax.axis_index("c")
    s = lax.axis_index("s")
    wid = s * NC + c
    pltpu.sync_copy(dst_hbm.at[wid], dst_v)

    zeros = jnp.zeros((16,), jnp.float32)

    def zero_body(i, carry):
        zbuf[pl.ds(i * 16, 16)] = zeros
        return carry

    lax.fori_loop(0, 1024 // 16, zero_body, 0)

    @pl.when(s < 10)
    def _():
        off = pl.multiple_of(s * 1000, 8)
        pltpu.sync_copy(zbuf.at[pl.ds(0, 1000)],
                        deg_sh.at[pl.ds(off, 1000)])

    plsc.subcore_barrier()

    def body(ci, carry):
        pltpu.sync_copy(w_hbm.at[wid, ci], wchunk_v)
        pltpu.sync_copy(wchunk_v, deg_sh.at[dst_v.at[ci]], add=True)
        return carry

    lax.fori_loop(0, NCHUNK, body, 0)
    plsc.subcore_barrier()

    @pl.when(s < 10)
    def _():
        off = pl.multiple_of(s * 1000, 8)
        oof = pl.multiple_of(c * N + s * 1000, 8)
        pltpu.sync_copy(deg_sh.at[pl.ds(off, 1000)],
                        zbuf.at[pl.ds(0, 1000)])
        pltpu.sync_copy(zbuf.at[pl.ds(0, 1000)],
                        out_hbm.at[pl.ds(oof, 1000)])


def _sc_deg(dst3, w3):
    k = functools.partial(
        pl.kernel,
        mesh=_mesh,
        out_type=jax.ShapeDtypeStruct((NC * N,), jnp.float32),
        scratch_types=[
            pltpu.VMEM((NCHUNK, CH), jnp.int32),
            pltpu.VMEM((CH,), jnp.float32),
            pltpu.VMEM((1024,), jnp.float32),
            pltpu.VMEM_SHARED((N,), jnp.float32),
        ],
    )(_sc_deg_body)
    return k(dst3, w3)


def _sc_edge_body(src_hbm, dst_hbm, w_hbm, hs_hbm, out_hbm,
                  dst_v, wchunk_v, src_v, rows_v, acc_sh, sem):
    c = lax.axis_index("c")
    s = lax.axis_index("s")
    wid = s * NC + c

    pltpu.sync_copy(dst_hbm.at[wid], dst_v)

    zeros = jnp.zeros((16,), jnp.float32)
    for i in range(CH):
        for j in range(D // 16):
            rows_v[i, pl.ds(j * 16, 16)] = zeros

    for k in range(8):
        ci = s + k * NS

        @pl.when(ci < NCHUNK)
        def _():
            off = pl.multiple_of(ci * CH, 8)
            pltpu.sync_copy(rows_v, acc_sh.at[pl.ds(off, CH)])

    plsc.subcore_barrier()

    def chunk_body(ci, carry):
        pltpu.sync_copy(src_hbm.at[wid, ci], src_v)
        pltpu.sync_copy(w_hbm.at[wid, ci], wchunk_v)
        pltpu.async_copy(hs_hbm.at[src_v], rows_v, sem).wait()
        for g in range(CH // 16):
            w_vec = wchunk_v[pl.ds(g * 16, 16)]
            for r in range(16):
                e = g * 16 + r
                ws = _splat(w_vec, r)
                for j in range(D // 16):
                    rows_v[e, pl.ds(j * 16, 16)] = (
                        rows_v[e, pl.ds(j * 16, 16)] * ws)
        pltpu.sync_copy(rows_v, acc_sh.at[dst_v.at[ci]], add=True)
        return carry

    lax.fori_loop(0, NCHUNK, chunk_body, 0)
    plsc.subcore_barrier()

    for k in range(8):
        ci = s + k * NS

        @pl.when(ci < NCHUNK)
        def _():
            off = pl.multiple_of(ci * CH, 8)
            oof = pl.multiple_of(c * N + ci * CH, 8)
            pltpu.sync_copy(acc_sh.at[pl.ds(off, CH)], rows_v)
            pltpu.sync_copy(rows_v, out_hbm.at[pl.ds(oof, CH)])


def _sc_edges(src3, dst3, w3, hs):
    k = functools.partial(
        pl.kernel,
        mesh=_mesh,
        out_type=jax.ShapeDtypeStruct((NC * N, D), jnp.float32),
        scratch_types=[
            pltpu.VMEM((NCHUNK, CH), jnp.int32),
            pltpu.VMEM((CH,), jnp.float32),
            pltpu.VMEM((CH,), jnp.int32),
            pltpu.VMEM((CH, D), jnp.float32),
            pltpu.VMEM_SHARED((N, D), jnp.float32),
            pltpu.SemaphoreType.DMA,
        ],
    )(_sc_edge_body)
    return k(src3, dst3, w3, hs)


_BLK = 1000


def _tc_mm_body(x_ref, w_ref, degp_ref, h_ref, hs_ref):
    xb = x_ref[...]
    h = jnp.dot(xb, w_ref[...], preferred_element_type=jnp.float32)
    deg = 1.0 + jnp.sum(degp_ref[...], axis=1, keepdims=True)
    dinv = lax.rsqrt(deg)
    h_ref[...] = h
    hs_ref[...] = h * dinv


def _tc_mm(x, W, degp_t):
    grid = (N // _BLK,)
    return pl.pallas_call(
        _tc_mm_body,
        grid=grid,
        in_specs=[
            pl.BlockSpec((_BLK, D), lambda i: (i, 0)),
            pl.BlockSpec((D, D), lambda i: (0, 0)),
            pl.BlockSpec((_BLK, NC), lambda i: (i, 0)),
        ],
        out_specs=[
            pl.BlockSpec((_BLK, D), lambda i: (i, 0)),
            pl.BlockSpec((_BLK, D), lambda i: (i, 0)),
        ],
        out_shape=[
            jax.ShapeDtypeStruct((N, D), jnp.float32),
            jax.ShapeDtypeStruct((N, D), jnp.float32),
        ],
    )(x, W, degp_t)


def _tc_final_body(acc_ref, h_ref, degp_ref, b_ref, g_ref, be_ref, o_ref):
    acc = acc_ref[0] + acc_ref[1]
    deg = 1.0 + jnp.sum(degp_ref[...], axis=1, keepdims=True)
    dinv = lax.rsqrt(deg)
    pre = acc * dinv + h_ref[...] * (dinv * dinv) + b_ref[...]
    mean = jnp.mean(pre, axis=0, keepdims=True)
    var = jnp.mean((pre - mean) * (pre - mean), axis=0, keepdims=True)
    o = (pre - mean) * lax.rsqrt(var + 1e-5) * g_ref[...] + be_ref[...]
    o_ref[...] = jnp.maximum(o, 0.0)


def _tc_final(acc, h, degp_t, b, gamma, beta):
    return pl.pallas_call(
        _tc_final_body,
        out_shape=jax.ShapeDtypeStruct((N, D), jnp.float32),
    )(acc, h, degp_t, b, gamma, beta)


def kernel(x, edge_index, edge_weight, W, b, gamma, beta):
    src = edge_index[0]
    dst = edge_index[1]
    src3 = src.reshape(NW, NCHUNK, CH)
    dst3 = dst.reshape(NW, NCHUNK, CH)
    w3 = edge_weight.reshape(NW, NCHUNK, CH)

    degp = _sc_deg(dst3, w3).reshape(NC, N)
    degp_t = degp.T
    h, hs = _tc_mm(x, W, degp_t)
    acc = _sc_edges(src3, dst3, w3, hs).reshape(NC, N, D)
    out = _tc_final(acc, h, degp_t,
                    b.reshape(1, D), gamma.reshape(1, D), beta.reshape(1, D))
    return out


# speedup vs baseline: 23.0535x; 1.4963x over previous
import functools

import jax
import jax.numpy as jnp
from jax import lax
from jax.experimental import pallas as pl
from jax.experimental.pallas import tpu as pltpu
from jax.experimental.pallas import tpu_sc as plsc

N = 10000
E = 320000
D = 128

NC = 2
NS = 16
NW = NC * NS
EPT = E // NW
CH = 80
NCHUNK = EPT // CH
RPT = N // NS

_mesh = plsc.VectorSubcoreMesh(core_axis_name="c", subcore_axis_name="s")

_GD = lax.GatherDimensionNumbers(
    offset_dims=(), collapsed_slice_dims=(0,), start_index_map=(0,))


def _splat(v, r):
    idx = jnp.full((16, 1), r, jnp.int32)
    return lax.gather(v, idx, _GD, (1,),
                      mode=lax.GatherScatterMode.PROMISE_IN_BOUNDS)


def _sc_deg_body(dst_hbm, w_hbm, out_hbm, dst_v, w_v, zbuf, deg_sh):
    c = lax.axis_index("c")
    s = lax.axis_index("s")
    wid = s * NC + c
    pltpu.sync_copy(dst_hbm.at[wid], dst_v)
    pltpu.sync_copy(w_hbm.at[wid, 0], w_v)

    zeros = jnp.zeros((16,), jnp.float32)

    def zero_body(i, carry):
        zbuf[pl.ds(i * 16, 16)] = zeros
        return carry

    lax.fori_loop(0, 1024 // 16, zero_body, 0)

    @pl.when(s < 10)
    def _():
        off = pl.multiple_of(s * 1000, 8)
        pltpu.sync_copy(zbuf.at[pl.ds(0, 1000)],
                        deg_sh.at[pl.ds(off, 1000)])

    plsc.subcore_barrier()

    def body(ci, carry):
        off = pl.multiple_of(ci * CH, 8)
        pltpu.sync_copy(w_v.at[pl.ds(off, CH)],
                        deg_sh.at[dst_v.at[ci]], add=True)
        return carry

    lax.fori_loop(0, NCHUNK, body, 0)
    plsc.subcore_barrier()

    @pl.when(s < 10)
    def _():
        off = pl.multiple_of(s * 1000, 8)
        oof = pl.multiple_of(c * N + s * 1000, 8)
        pltpu.sync_copy(deg_sh.at[pl.ds(off, 1000)],
                        zbuf.at[pl.ds(0, 1000)])
        pltpu.sync_copy(zbuf.at[pl.ds(0, 1000)],
                        out_hbm.at[pl.ds(oof, 1000)])


def _sc_deg(dst3, w2):
    k = functools.partial(
        pl.kernel,
        mesh=_mesh,
        out_type=jax.ShapeDtypeStruct((NC * N,), jnp.float32),
        scratch_types=[
            pltpu.VMEM((NCHUNK, CH), jnp.int32),
            pltpu.VMEM((EPT,), jnp.float32),
            pltpu.VMEM((1024,), jnp.float32),
            pltpu.VMEM_SHARED((N,), jnp.float32),
        ],
    )(_sc_deg_body)
    return k(dst3, w2)


def _sc_edge_body(src_hbm, dst_hbm, w_hbm, hs_hbm, out_hbm,
                  dst_v, src_v, w_v, rows0, rows1, rows2,
                  acc_sh, g0, g1, g2, s0, s1, s2, t0, t1, t2):
    c = lax.axis_index("c")
    s = lax.axis_index("s")
    wid = s * NC + c
    rows = (rows0, rows1, rows2)
    gsem = (g0, g1, g2)
    ssem = (s0, s1, s2)
    tsem = (t0, t1, t2)

    zeros = jnp.zeros((16,), jnp.float32)
    for i in range(CH):
        for j in range(D // 16):
            rows0[i, pl.ds(j * 16, 16)] = zeros

    for k in range(8):
        zi = s + k * NS

        @pl.when(zi < NCHUNK)
        def _():
            off = pl.multiple_of(zi * CH, 8)
            pltpu.sync_copy(rows0, acc_sh.at[pl.ds(off, CH)])

    def fire_stage(ci, b):
        pltpu.async_copy(src_hbm.at[wid, ci], src_v.at[b], tsem[b])
        pltpu.async_copy(dst_hbm.at[wid, ci], dst_v.at[b], tsem[b])
        pltpu.async_copy(w_hbm.at[wid, ci], w_v.at[b], tsem[b])

    def wait_stage(b):
        pltpu.make_async_copy(src_hbm.at[0, 0], src_v.at[b], tsem[b]).wait()
        pltpu.make_async_copy(dst_hbm.at[0, 0], dst_v.at[b], tsem[b]).wait()
        pltpu.make_async_copy(w_hbm.at[0, 0], w_v.at[b], tsem[b]).wait()

    def fire_gather(b):
        pltpu.async_copy(hs_hbm.at[src_v.at[b]], rows[b], gsem[b])

    def wait_gather(b):
        pltpu.make_async_copy(hs_hbm.at[src_v.at[b]], rows[b],
                              gsem[b]).wait()

    def scale(b):
        rv = rows[b]
        for g in range(CH // 16):
            w_vec = w_v[b, pl.ds(g * 16, 16)]
            for r in range(16):
                e = g * 16 + r
                ws = _splat(w_vec, r)
                for j in range(D // 16):
                    rv[e, pl.ds(j * 16, 16)] = rv[e, pl.ds(j * 16, 16)] * ws

    def phase(ci, b):
        bp = (b + 1) % 3
        bn = (b + 2) % 3

        @pl.when(ci + 1 < NCHUNK)
        def _():
            wait_stage(bp)
            fire_gather(bp)

        wait_gather(b)
        scale(b)
        pltpu.async_copy(rows[b], acc_sh.at[dst_v.at[b]],
                         ssem[b], add=True).wait()

        @pl.when(ci + 2 < NCHUNK)
        def _():
            fire_stage(ci + 2, bn)

    fire_stage(0, 0)
    fire_stage(1, 1)
    plsc.subcore_barrier()
    wait_stage(0)
    fire_gather(0)

    def triple(k, carry):
        ci = k * 3
        phase(ci, 0)
        phase(ci + 1, 1)
        phase(ci + 2, 2)
        return carry

    lax.fori_loop(0, NCHUNK // 3, triple, 0)
    phase(NCHUNK - 2, 0)
    phase(NCHUNK - 1, 1)
    plsc.subcore_barrier()

    for k in range(8):
        ci = s + k * NS

        @pl.when(ci < NCHUNK)
        def _():
            off = pl.multiple_of(ci * CH, 8)
            oof = pl.multiple_of(c * N + ci * CH, 8)
            pltpu.sync_copy(acc_sh.at[pl.ds(off, CH)], rows0)
            pltpu.sync_copy(rows0, out_hbm.at[pl.ds(oof, CH)])


def _sc_edges(src2, dst3, w2, hs):
    k = functools.partial(
        pl.kernel,
        mesh=_mesh,
        out_type=jax.ShapeDtypeStruct((NC * N, D), jnp.float32),
        scratch_types=[
            pltpu.VMEM((3, CH), jnp.int32),
            pltpu.VMEM((3, CH), jnp.int32),
            pltpu.VMEM((3, CH), jnp.float32),
            pltpu.VMEM((CH, D), jnp.float32),
            pltpu.VMEM((CH, D), jnp.float32),
            pltpu.VMEM((CH, D), jnp.float32),
            pltpu.VMEM_SHARED((N, D), jnp.float32),
        ] + [pltpu.SemaphoreType.DMA] * 9,
    )(_sc_edge_body)
    return k(src2, dst3, w2, hs)


_BLK = 1000


def _tc_mm_body(x_ref, w_ref, degp_ref, h_ref, hs_ref):
    xb = x_ref[...]
    h = jnp.dot(xb, w_ref[...], preferred_element_type=jnp.float32)
    deg = 1.0 + jnp.sum(degp_ref[...], axis=1, keepdims=True)
    dinv = lax.rsqrt(deg)
    h_ref[...] = h
    hs_ref[...] = h * dinv


def _tc_mm(x, W, degp_t):
    grid = (N // _BLK,)
    return pl.pallas_call(
        _tc_mm_body,
        grid=grid,
        in_specs=[
            pl.BlockSpec((_BLK, D), lambda i: (i, 0)),
            pl.BlockSpec((D, D), lambda i: (0, 0)),
            pl.BlockSpec((_BLK, NC), lambda i: (i, 0)),
        ],
        out_specs=[
            pl.BlockSpec((_BLK, D), lambda i: (i, 0)),
            pl.BlockSpec((_BLK, D), lambda i: (i, 0)),
        ],
        out_shape=[
            jax.ShapeDtypeStruct((N, D), jnp.float32),
            jax.ShapeDtypeStruct((N, D), jnp.float32),
        ],
    )(x, W, degp_t)


def _tc_final_body(acc_ref, h_ref, degp_ref, b_ref, g_ref, be_ref, o_ref):
    acc = acc_ref[0] + acc_ref[1]
    deg = 1.0 + jnp.sum(degp_ref[...], axis=1, keepdims=True)
    dinv = lax.rsqrt(deg)
    pre = acc * dinv + h_ref[...] * (dinv * dinv) + b_ref[...]
    mean = jnp.mean(pre, axis=0, keepdims=True)
    var = jnp.mean((pre - mean) * (pre - mean), axis=0, keepdims=True)
    o = (pre - mean) * lax.rsqrt(var + 1e-5) * g_ref[...] + be_ref[...]
    o_ref[...] = jnp.maximum(o, 0.0)


def _tc_final(acc, h, degp_t, b, gamma, beta):
    return pl.pallas_call(
        _tc_final_body,
        out_shape=jax.ShapeDtypeStruct((N, D), jnp.float32),
    )(acc, h, degp_t, b, gamma, beta)


def kernel(x, edge_index, edge_weight, W, b, gamma, beta):
    src = edge_index[0]
    dst = edge_index[1]
    src3 = src.reshape(NW, NCHUNK, CH)
    dst3 = dst.reshape(NW, NCHUNK, CH)
    w3 = edge_weight.reshape(NW, NCHUNK, CH)
    w2 = edge_weight.reshape(NW, 1, EPT)

    degp = _sc_deg(dst3, w2).reshape(NC, N)
    degp_t = degp.T
    h, hs = _tc_mm(x, W, degp_t)
    acc = _sc_edges(src3, dst3, w3, hs).reshape(NC, N, D)
    out = _tc_final(acc, h, degp_t,
                    b.reshape(1, D), gamma.reshape(1, D), beta.reshape(1, D))
    return out


# speedup vs baseline: 28.4415x; 1.2337x over previous
import functools

import jax
import jax.numpy as jnp
from jax import lax
from jax.experimental import pallas as pl
from jax.experimental.pallas import tpu as pltpu
from jax.experimental.pallas import tpu_sc as plsc

N = 10000
E = 320000
D = 128

NC = 2
NS = 16
NW = NC * NS
EPT = E // NW
CH = 80
NCHUNK = EPT // CH
RPT = N // NS

_mesh = plsc.VectorSubcoreMesh(core_axis_name="c", subcore_axis_name="s")

_GD = lax.GatherDimensionNumbers(
    offset_dims=(), collapsed_slice_dims=(0,), start_index_map=(0,))


def _splat(v, r):
    idx = jnp.full((16, 1), r, jnp.int32)
    return lax.gather(v, idx, _GD, (1,),
                      mode=lax.GatherScatterMode.PROMISE_IN_BOUNDS)


def _sc_deg_body(dst_hbm, w_hbm, out_hbm, dst_v, w_v, zbuf, deg_sh):
    c = lax.axis_index("c")
    s = lax.axis_index("s")
    wid = s * NC + c
    pltpu.sync_copy(dst_hbm.at[wid], dst_v)
    pltpu.sync_copy(w_hbm.at[wid, 0], w_v)

    zeros = jnp.zeros((16,), jnp.float32)

    def zero_body(i, carry):
        zbuf[pl.ds(i * 16, 16)] = zeros
        return carry

    lax.fori_loop(0, 1024 // 16, zero_body, 0)

    @pl.when(s < 10)
    def _():
        off = pl.multiple_of(s * 1000, 8)
        pltpu.sync_copy(zbuf.at[pl.ds(0, 1000)],
                        deg_sh.at[pl.ds(off, 1000)])

    plsc.subcore_barrier()

    def body(ci, carry):
        off = pl.multiple_of(ci * CH, 8)
        pltpu.sync_copy(w_v.at[pl.ds(off, CH)],
                        deg_sh.at[dst_v.at[ci]], add=True)
        return carry

    lax.fori_loop(0, NCHUNK, body, 0)
    plsc.subcore_barrier()

    @pl.when(s < 10)
    def _():
        off = pl.multiple_of(s * 1000, 8)
        oof = pl.multiple_of(c * N + s * 1000, 8)
        pltpu.sync_copy(deg_sh.at[pl.ds(off, 1000)],
                        zbuf.at[pl.ds(0, 1000)])
        pltpu.sync_copy(zbuf.at[pl.ds(0, 1000)],
                        out_hbm.at[pl.ds(oof, 1000)])


def _sc_deg(dst3, w2):
    k = functools.partial(
        pl.kernel,
        mesh=_mesh,
        out_type=jax.ShapeDtypeStruct((NC * N,), jnp.float32),
        scratch_types=[
            pltpu.VMEM((NCHUNK, CH), jnp.int32),
            pltpu.VMEM((EPT,), jnp.float32),
            pltpu.VMEM((1024,), jnp.float32),
            pltpu.VMEM_SHARED((N,), jnp.float32),
        ],
    )(_sc_deg_body)
    return k(dst3, w2)


def _sc_edge_body(src_hbm, dst_hbm, w_hbm, hs_hbm, out_hbm,
                  dst_v, src_v, w_v, rin0, rin1, rout,
                  acc_sh, g0, g1, t0, t1, ssem):
    c = lax.axis_index("c")
    s = lax.axis_index("s")
    wid = s * NC + c
    rin = (rin0, rin1)
    gsem = (g0, g1)
    tsem = (t0, t1)

    zeros = jnp.zeros((16,), jnp.float32)
    for i in range(CH):
        for j in range(D // 16):
            rout[i, pl.ds(j * 16, 16)] = zeros

    for k in range(8):
        zi = s + k * NS

        @pl.when(zi < NCHUNK)
        def _():
            off = pl.multiple_of(zi * CH, 8)
            pltpu.sync_copy(rout, acc_sh.at[pl.ds(off, CH)])

    def fire_stage(ci, b):
        pltpu.async_copy(src_hbm.at[wid, ci], src_v.at[b], tsem[b])
        pltpu.async_copy(dst_hbm.at[wid, ci], dst_v.at[b], tsem[b])
        pltpu.async_copy(w_hbm.at[wid, ci], w_v.at[b], tsem[b])

    def wait_stage(b):
        pltpu.make_async_copy(src_hbm.at[0, 0], src_v.at[b], tsem[b]).wait()
        pltpu.make_async_copy(dst_hbm.at[0, 0], dst_v.at[b], tsem[b]).wait()
        pltpu.make_async_copy(w_hbm.at[0, 0], w_v.at[b], tsem[b]).wait()

    def fire_gather(b):
        pltpu.async_copy(hs_hbm.at[src_v.at[b]], rin[b], gsem[b])

    def wait_gather(b):
        pltpu.make_async_copy(hs_hbm.at[src_v.at[b]], rin[b],
                              gsem[b]).wait()

    def scale(b):
        rv = rin[b]
        for g in range(CH // 16):
            w_vec = w_v[b, pl.ds(g * 16, 16)]
            for r in range(16):
                e = g * 16 + r
                ws = _splat(w_vec, r)
                for j in range(D // 16):
                    rout[e, pl.ds(j * 16, 16)] = (
                        rv[e, pl.ds(j * 16, 16)] * ws)

    def phase(ci, b):
        bp = (b + 1) % 2

        @pl.when(ci + 1 < NCHUNK)
        def _():
            wait_stage(bp)
            fire_gather(bp)

        wait_gather(b)
        scale(b)
        pltpu.async_copy(rout, acc_sh.at[dst_v.at[b]],
                         ssem, add=True).wait()

        @pl.when(ci + 2 < NCHUNK)
        def _():
            fire_stage(ci + 2, b)

    fire_stage(0, 0)
    fire_stage(1, 1)
    plsc.subcore_barrier()
    wait_stage(0)
    fire_gather(0)

    def double(k, carry):
        ci = k * 2
        phase(ci, 0)
        phase(ci + 1, 1)
        return carry

    lax.fori_loop(0, NCHUNK // 2, double, 0)
    phase(NCHUNK - 1, 0)
    plsc.subcore_barrier()

    for k in range(8):
        ci = s + k * NS

        @pl.when(ci < NCHUNK)
        def _():
            off = pl.multiple_of(ci * CH, 8)
            oof = pl.multiple_of(c * N + ci * CH, 8)
            pltpu.sync_copy(acc_sh.at[pl.ds(off, CH)], rout)
            pltpu.sync_copy(rout, out_hbm.at[pl.ds(oof, CH)])


def _sc_edges(src2, dst3, w2, hs):
    k = functools.partial(
        pl.kernel,
        mesh=_mesh,
        out_type=jax.ShapeDtypeStruct((NC * N, D), jnp.float32),
        scratch_types=[
            pltpu.VMEM((2, CH), jnp.int32),
            pltpu.VMEM((2, CH), jnp.int32),
            pltpu.VMEM((2, CH), jnp.float32),
            pltpu.VMEM((CH, D), jnp.float32),
            pltpu.VMEM((CH, D), jnp.float32),
            pltpu.VMEM((CH, D), jnp.float32),
            pltpu.VMEM_SHARED((N, D), jnp.float32),
        ] + [pltpu.SemaphoreType.DMA] * 5,
    )(_sc_edge_body)
    return k(src2, dst3, w2, hs)


_BLK = 1000


def _tc_mm_body(x_ref, w_ref, degp_ref, h_ref, hs_ref):
    xb = x_ref[...]
    h = jnp.dot(xb, w_ref[...], preferred_element_type=jnp.float32)
    deg = 1.0 + jnp.sum(degp_ref[...], axis=1, keepdims=True)
    dinv = lax.rsqrt(deg)
    h_ref[...] = h
    hs_ref[...] = h * dinv


def _tc_mm(x, W, degp_t):
    grid = (N // _BLK,)
    return pl.pallas_call(
        _tc_mm_body,
        grid=grid,
        in_specs=[
            pl.BlockSpec((_BLK, D), lambda i: (i, 0)),
            pl.BlockSpec((D, D), lambda i: (0, 0)),
            pl.BlockSpec((_BLK, NC), lambda i: (i, 0)),
        ],
        out_specs=[
            pl.BlockSpec((_BLK, D), lambda i: (i, 0)),
            pl.BlockSpec((_BLK, D), lambda i: (i, 0)),
        ],
        out_shape=[
            jax.ShapeDtypeStruct((N, D), jnp.float32),
            jax.ShapeDtypeStruct((N, D), jnp.float32),
        ],
    )(x, W, degp_t)


def _tc_final_body(acc_ref, h_ref, degp_ref, b_ref, g_ref, be_ref, o_ref):
    acc = acc_ref[0] + acc_ref[1]
    deg = 1.0 + jnp.sum(degp_ref[...], axis=1, keepdims=True)
    dinv = lax.rsqrt(deg)
    pre = acc * dinv + h_ref[...] * (dinv * dinv) + b_ref[...]
    mean = jnp.mean(pre, axis=0, keepdims=True)
    var = jnp.mean((pre - mean) * (pre - mean), axis=0, keepdims=True)
    o = (pre - mean) * lax.rsqrt(var + 1e-5) * g_ref[...] + be_ref[...]
    o_ref[...] = jnp.maximum(o, 0.0)


def _tc_final(acc, h, degp_t, b, gamma, beta):
    return pl.pallas_call(
        _tc_final_body,
        out_shape=jax.ShapeDtypeStruct((N, D), jnp.float32),
    )(acc, h, degp_t, b, gamma, beta)


def kernel(x, edge_index, edge_weight, W, b, gamma, beta):
    src = edge_index[0]
    dst = edge_index[1]
    src3 = src.reshape(NW, NCHUNK, CH)
    dst3 = dst.reshape(NW, NCHUNK, CH)
    w3 = edge_weight.reshape(NW, NCHUNK, CH)
    w2 = edge_weight.reshape(NW, 1, EPT)

    degp = _sc_deg(dst3, w2).reshape(NC, N)
    degp_t = degp.T
    h, hs = _tc_mm(x, W, degp_t)
    acc = _sc_edges(src3, dst3, w3, hs).reshape(NC, N, D)
    out = _tc_final(acc, h, degp_t,
                    b.reshape(1, D), gamma.reshape(1, D), beta.reshape(1, D))
    return out
